# baseline (speedup 1.0000x reference)
import jax
import jax.numpy as jnp
from jax import lax
from jax.experimental import pallas as pl
from jax.experimental.pallas import tpu as pltpu
from jax.experimental.pallas import tpu_sc as plsc

NC = 2
NS = 16
NW = NC * NS
L = 16
D = 32
SUB = 128


def _sc_body(C, NSUB,
             ent_w, rel_w, tem_w,
             pos_h, pos_r, pos_tem, pos_t,
             neg_h, neg_r, neg_tem, neg_t,
             pos_out, neg_out,
             idx_v, rows_h, rows_r, rows_m, rows_t, out_v, sem):
    rows_v = (rows_h, rows_r, rows_m, rows_t)
    wid = lax.axis_index("s") * NC + lax.axis_index("c")
    base = wid * C
    tables = (ent_w, rel_w, tem_w, ent_w)
    sides = (
        ((pos_h, pos_r, pos_tem, pos_t), pos_out),
        ((neg_h, neg_r, neg_tem, neg_t), neg_out),
    )
    for idx_hbm, out_hbm in sides:
        cps = []
        for t in range(4):
            for k in range(NSUB):
                cps.append(pltpu.async_copy(
                    idx_hbm[t].at[pl.ds(base + k * SUB, SUB)],
                    idx_v.at[t, k], sem))
        for c in cps:
            c.wait()
        cps = []
        for t in range(4):
            for k in range(NSUB):
                cps.append(pltpu.async_copy(
                    tables[t].at[idx_v.at[t, k]],
                    rows_v[t].at[pl.ds(k * SUB, SUB)], sem))
        for c in cps:
            c.wait()

        def group_step(g, carry):
            rows = g * L + lax.iota(jnp.int32, L)

            def col_step(j, acc):
                cj = jnp.full((L,), 0, jnp.int32) + j
                h = plsc.load_gather(rows_h, [rows, cj])
                r = plsc.load_gather(rows_r, [rows, cj])
                m = plsc.load_gather(rows_m, [rows, cj])
                tt = plsc.load_gather(rows_t, [rows, cj])
                return acc + jnp.abs(h + r + m - tt)

            acc = lax.fori_loop(0, D, col_step, jnp.zeros((L,), jnp.float32))
            out_v[pl.ds(g * L, L)] = acc
            return carry

        lax.fori_loop(0, C // L, group_step, 0)
        pltpu.sync_copy(out_v, out_hbm.at[pl.ds(base, C)])


def kernel(pos_h, pos_t, pos_r, pos_tem, neg_h, neg_t, neg_r, neg_tem,
           ent_w, rel_w, tem_w):
    B = pos_h.shape[0]
    C = B // NW
    NSUB = C // SUB
    mesh = plsc.VectorSubcoreMesh(core_axis_name="c", subcore_axis_name="s")

    def body(*refs):
        _sc_body(C, NSUB, *refs)

    f = pl.kernel(
        body,
        out_type=(jax.ShapeDtypeStruct((B,), jnp.float32),
                  jax.ShapeDtypeStruct((B,), jnp.float32)),
        mesh=mesh,
        scratch_types=[
            pltpu.VMEM((4, NSUB, SUB), jnp.int32),
            pltpu.VMEM((C, D), jnp.float32),
            pltpu.VMEM((C, D), jnp.float32),
            pltpu.VMEM((C, D), jnp.float32),
            pltpu.VMEM((C, D), jnp.float32),
            pltpu.VMEM((C,), jnp.float32),
            pltpu.SemaphoreType.DMA,
        ],
        compiler_params=pltpu.CompilerParams(
            needs_layout_passes=False, use_tc_tiling_on_sc=False),
    )
    i32 = jnp.int32
    return f(ent_w, rel_w, tem_w,
             pos_h.astype(i32), pos_r.astype(i32), pos_tem.astype(i32),
             pos_t.astype(i32),
             neg_h.astype(i32), neg_r.astype(i32), neg_tem.astype(i32),
             neg_t.astype(i32))
